# Initial kernel scaffold; baseline (speedup 1.0000x reference)
import jax
import jax.numpy as jnp
from jax.experimental import pallas as pl


def kernel(x, edge_index, batch, W_emb, b_emb, W_f, b_f, W_s, b_s, W_fc, b_fc, W_out, b_out):
    raise NotImplementedError("write your pallas kernel here")


# speedup vs baseline: 3.9793x; 3.9793x over previous
import functools

import jax
import jax.numpy as jnp
from jax import lax
from jax.experimental import pallas as pl
from jax.experimental.pallas import tpu as pltpu
from jax.experimental.pallas import tpu_sc as plsc

F32 = jnp.float32

_LOG1P_Q = (0.9999016475119358, -0.4978750019864502, 0.31764828238151116,
            -0.19375874124753323, 0.08556748448678754, -0.018337923604283612)

_NC, _NS, _LANES = 2, 16, 16
_NW = _NC * _NS


def _emb_body(x_ref, wemb_ref, bemb_ref, t_ref, u_ref, bias_ref,
              h_ref, a_ref, b_ref):
    h = jnp.dot(x_ref[...], wemb_ref[...], preferred_element_type=F32)
    h = h + bemb_ref[...]
    h_ref[...] = h
    a_ref[...] = jnp.dot(h, t_ref[...], preferred_element_type=F32) + bias_ref[...]
    b_ref[...] = jnp.dot(h, u_ref[...], preferred_element_type=F32)


def _layer_body(h_ref, p0_ref, p1_ref, t_ref, u_ref, bias_ref,
                hn_ref, a_ref, b_ref):
    hn = h_ref[...] + p0_ref[...] + p1_ref[...]
    hn_ref[...] = hn
    a_ref[...] = jnp.dot(hn, t_ref[...], preferred_element_type=F32) + bias_ref[...]
    b_ref[...] = jnp.dot(hn, u_ref[...], preferred_element_type=F32)


def _readout_body(h_ref, p0_ref, p1_ref, batch_ref, wfc_ref, bfc_ref,
                  wout_ref, bout_ref, sums_ref, cnt_ref, out_ref, *, nblk, g):
    i = pl.program_id(0)

    @pl.when(i == 0)
    def _():
        sums_ref[...] = jnp.zeros_like(sums_ref)
        cnt_ref[...] = jnp.zeros_like(cnt_ref)

    hf = h_ref[...] + p0_ref[...] + p1_ref[...]
    bb = batch_ref[0]
    gid = lax.broadcasted_iota(jnp.int32, (g, bb.shape[-1]), 0)
    mask = (jnp.broadcast_to(bb, (g, bb.shape[-1])) == gid).astype(F32)
    sums_ref[...] += jnp.dot(mask, hf, preferred_element_type=F32)
    cnt_ref[...] += jnp.broadcast_to(
        jnp.sum(mask, axis=1, keepdims=True), cnt_ref.shape)

    @pl.when(i == nblk - 1)
    def _():
        pooled = sums_ref[...] / jnp.maximum(cnt_ref[...], 1.0)
        fc = jax.nn.softplus(
            jnp.dot(pooled, wfc_ref[...], preferred_element_type=F32)
            + bfc_ref[...])
        out_ref[...] = (jnp.dot(fc, wout_ref[...], preferred_element_type=F32)
                        + bout_ref[...])


def _sigmoid_softplus(uf, us):
    ef = jnp.exp(-jnp.abs(uf))
    p = 1.0 / (1.0 + ef)
    gate = jnp.where(uf >= 0, p, 1.0 - p)
    t = jnp.exp(-jnp.abs(us))
    q = jnp.full_like(t, _LOG1P_Q[5])
    for c in _LOG1P_Q[4::-1]:
        q = q * t + c
    sp = jnp.maximum(us, 0.0) + t * q
    return gate * sp


def _edge_body(a_hbm, b_hbm, dst_hbm, src_hbm, out_hbm,
               dsti, srci, arows, brows, mbuf, agg_sh, zbuf,
               sem_a, sem_b, *, epw, ch, nchunk, zrows, d):
    c = lax.axis_index("c")
    s = lax.axis_index("s")
    w = c * _NS + s
    nrows = agg_sh.shape[0] // _NS

    def _zrow(r, _):
        for v in range(d // _LANES):
            zbuf[r, pl.ds(v * _LANES, _LANES)] = jnp.zeros((_LANES,), F32)
        return 0
    lax.fori_loop(0, zrows, _zrow, 0)
    for k in range(nrows // zrows):
        pltpu.sync_copy(zbuf, agg_sh.at[pl.ds(s * nrows + k * zrows, zrows)])
    plsc.subcore_barrier()

    def _chunk(i, _):
        base = w * epw + i * ch
        pltpu.sync_copy(dst_hbm.at[pl.ds(base, ch)], dsti)
        pltpu.sync_copy(src_hbm.at[pl.ds(base, ch)], srci)
        cp_a = pltpu.async_copy(a_hbm.at[dsti], arows, sem_a)
        cp_b = pltpu.async_copy(b_hbm.at[srci], brows, sem_b)
        cp_a.wait()
        cp_b.wait()

        def _edge(e, _):
            for v in range(d // _LANES):
                uf = (arows[e, pl.ds(v * _LANES, _LANES)]
                      + brows[e, pl.ds(v * _LANES, _LANES)])
                us = (arows[e, pl.ds(d + v * _LANES, _LANES)]
                      + brows[e, pl.ds(d + v * _LANES, _LANES)])
                mbuf[e, pl.ds(v * _LANES, _LANES)] = _sigmoid_softplus(uf, us)
            return 0
        lax.fori_loop(0, ch, _edge, 0)

        pltpu.sync_copy(mbuf, agg_sh.at[dsti], add=True)
        return 0
    lax.fori_loop(0, nchunk, _chunk, 0)

    plsc.subcore_barrier()
    pltpu.sync_copy(agg_sh.at[pl.ds(s * nrows, nrows)],
                    out_hbm.at[c, pl.ds(s * nrows, nrows)])


def kernel(x, edge_index, batch, W_emb, b_emb, W_f, b_f, W_s, b_s,
           W_fc, b_fc, W_out, b_out):
    n, d_in = x.shape
    d = W_emb.shape[1]
    num_layers = W_f.shape[0]
    h_dim = W_fc.shape[1]
    g = 64
    e = edge_index.shape[1]

    src = edge_index[0]
    dst = edge_index[1]

    T = jnp.concatenate([W_f[:, :d, :], W_s[:, :d, :]], axis=2)
    U = jnp.concatenate([W_f[:, d:, :], W_s[:, d:, :]], axis=2)
    bias = jnp.concatenate([b_f, b_s], axis=1)

    bn = 2000
    nblk = n // bn

    def _full(shape):
        return pl.BlockSpec(shape, lambda i: tuple(0 for _ in shape))

    emb_call = pl.pallas_call(
        _emb_body,
        grid=(nblk,),
        in_specs=[
            pl.BlockSpec((bn, d_in), lambda i: (i, 0)),
            _full((d_in, d)), _full((1, d)),
            _full((d, 2 * d)), _full((d, 2 * d)), _full((1, 2 * d)),
        ],
        out_specs=[
            pl.BlockSpec((bn, d), lambda i: (i, 0)),
            pl.BlockSpec((bn, 2 * d), lambda i: (i, 0)),
            pl.BlockSpec((bn, 2 * d), lambda i: (i, 0)),
        ],
        out_shape=[
            jax.ShapeDtypeStruct((n, d), F32),
            jax.ShapeDtypeStruct((n, 2 * d), F32),
            jax.ShapeDtypeStruct((n, 2 * d), F32),
        ],
    )

    layer_call = pl.pallas_call(
        _layer_body,
        grid=(nblk,),
        in_specs=[
            pl.BlockSpec((bn, d), lambda i: (i, 0)),
            pl.BlockSpec((bn, d), lambda i: (i, 0)),
            pl.BlockSpec((bn, d), lambda i: (i, 0)),
            _full((d, 2 * d)), _full((d, 2 * d)), _full((1, 2 * d)),
        ],
        out_specs=[
            pl.BlockSpec((bn, d), lambda i: (i, 0)),
            pl.BlockSpec((bn, 2 * d), lambda i: (i, 0)),
            pl.BlockSpec((bn, 2 * d), lambda i: (i, 0)),
        ],
        out_shape=[
            jax.ShapeDtypeStruct((n, d), F32),
            jax.ShapeDtypeStruct((n, 2 * d), F32),
            jax.ShapeDtypeStruct((n, 2 * d), F32),
        ],
    )

    epw = e // _NW
    ch = 80
    nchunk = epw // ch
    n_pad = ((n + 8 * _NS - 1) // (8 * _NS)) * (8 * _NS)
    zrows = (n_pad // _NS) // 5
    mesh = plsc.VectorSubcoreMesh(core_axis_name="c", subcore_axis_name="s")
    edge_call = pl.kernel(
        functools.partial(_edge_body, epw=epw, ch=ch, nchunk=nchunk,
                          zrows=zrows, d=d),
        out_type=jax.ShapeDtypeStruct((_NC, n_pad, d), F32),
        mesh=mesh,
        scratch_types=[
            pltpu.VMEM((ch,), jnp.int32),
            pltpu.VMEM((ch,), jnp.int32),
            pltpu.VMEM((ch, 2 * d), F32),
            pltpu.VMEM((ch, 2 * d), F32),
            pltpu.VMEM((ch, d), F32),
            pltpu.VMEM_SHARED((n_pad, d), F32),
            pltpu.VMEM((zrows, d), F32),
            pltpu.SemaphoreType.DMA,
            pltpu.SemaphoreType.DMA,
        ],
    )

    readout_call = pl.pallas_call(
        functools.partial(_readout_body, nblk=nblk, g=g),
        grid=(nblk,),
        in_specs=[
            pl.BlockSpec((bn, d), lambda i: (i, 0)),
            pl.BlockSpec((bn, d), lambda i: (i, 0)),
            pl.BlockSpec((bn, d), lambda i: (i, 0)),
            pl.BlockSpec((1, 1, bn), lambda i: (i, 0, 0)),
            _full((d, h_dim)), _full((1, h_dim)),
            _full((h_dim, 128)), _full((1, 128)),
        ],
        out_specs=[
            pl.BlockSpec((g, d), lambda i: (0, 0)),
            pl.BlockSpec((g, d), lambda i: (0, 0)),
            pl.BlockSpec((g, 128), lambda i: (0, 0)),
        ],
        out_shape=[
            jax.ShapeDtypeStruct((g, d), F32),
            jax.ShapeDtypeStruct((g, d), F32),
            jax.ShapeDtypeStruct((g, 128), F32),
        ],
    )

    h, A, B = emb_call(x, W_emb, b_emb.reshape(1, d),
                       T[0], U[0], bias[0].reshape(1, 2 * d))
    for l in range(num_layers):
        parts = edge_call(A, B, dst, src)
        parts = parts[:, :n]
        if l + 1 < num_layers:
            h, A, B = layer_call(h, parts[0], parts[1],
                                 T[l + 1], U[l + 1],
                                 bias[l + 1].reshape(1, 2 * d))
        else:
            w_out_pad = jnp.pad(W_out, ((0, 0), (0, 128 - W_out.shape[1])))
            b_out_pad = jnp.pad(b_out, (0, 128 - b_out.shape[0])).reshape(1, 128)
            _, _, out128 = readout_call(
                h, parts[0], parts[1], batch.reshape(nblk, 1, bn),
                W_fc, b_fc.reshape(1, h_dim), w_out_pad, b_out_pad)
            return out128[:, :W_out.shape[1]]
